# baseline (speedup 1.0000x reference)
import jax
import jax.numpy as jnp
from jax import lax
from jax.experimental import pallas as pl
from jax.experimental.pallas import tpu as pltpu
from jax.experimental.pallas import tpu_sc as plsc

_N = 10000
_E = 320000
_D = 128
_H = 128
_C = 3
_G = 64
_EPS = 1e-5

_NSUB = 16
_NCORE = 2
_NW = _NSUB * _NCORE
_CHUNK = 128
_CPW = 80
_EPAD = _NW * _CPW * _CHUNK
_NPAD = 10240
_RPS = _NPAD // _NSUB


def _sc_degree_body(dst_hbm, out_hbm, idx_d, ones_v, zbuf, acc, sem):
    c = lax.axis_index("c")
    s = lax.axis_index("s")
    wid = c * _NSUB + s
    one16 = jnp.ones((16,), jnp.float32)
    zero16 = jnp.zeros((16,), jnp.float32)
    for r in range(16):
        zbuf[r] = zero16
    for r in range(_CHUNK):
        ones_v[r] = one16

    def zloop(i, carry):
        pltpu.sync_copy(zbuf, acc.at[pl.ds(s * _RPS + i * 16, 16)])
        return carry
    lax.fori_loop(0, _RPS // 16, zloop, 0)
    plsc.subcore_barrier()

    def eloop(i, carry):
        base = (wid * _CPW + i) * _CHUNK
        pltpu.sync_copy(dst_hbm.at[pl.ds(base, _CHUNK)], idx_d)
        pltpu.sync_copy(ones_v, acc.at[idx_d], add=True)
        return carry
    lax.fori_loop(0, _CPW, eloop, 0)
    plsc.subcore_barrier()

    pltpu.sync_copy(acc.at[pl.ds(s * _RPS, _RPS)],
                    out_hbm.at[c, pl.ds(s * _RPS, _RPS)])


_sc_degree = pl.kernel(
    _sc_degree_body,
    out_type=jax.ShapeDtypeStruct((_NCORE, _NPAD, 16), jnp.float32),
    mesh=plsc.VectorSubcoreMesh(core_axis_name="c", subcore_axis_name="s"),
    scratch_types=[
        pltpu.VMEM((_CHUNK,), jnp.int32),
        pltpu.VMEM((_CHUNK, 16), jnp.float32),
        pltpu.VMEM((16, 16), jnp.float32),
        pltpu.VMEM_SHARED((_NPAD, 16), jnp.float32),
        pltpu.SemaphoreType.DMA,
    ],
)


def _sc_scatter_body(xs_hbm, src_hbm, dst_hbm, out_hbm,
                     idx_s, idx_d, rows_v, zbuf, acc, sem):
    c = lax.axis_index("c")
    s = lax.axis_index("s")
    wid = c * _NSUB + s
    zero16 = jnp.zeros((16,), jnp.float32)
    for r in range(16):
        for j in range(_H // 16):
            zbuf[r, pl.ds(j * 16, 16)] = zero16

    def zloop(i, carry):
        pltpu.sync_copy(zbuf, acc.at[pl.ds(s * _RPS + i * 16, 16)])
        return carry
    lax.fori_loop(0, _RPS // 16, zloop, 0)
    plsc.subcore_barrier()

    def eloop(i, carry):
        base = (wid * _CPW + i) * _CHUNK
        pltpu.sync_copy(src_hbm.at[pl.ds(base, _CHUNK)], idx_s)
        pltpu.sync_copy(dst_hbm.at[pl.ds(base, _CHUNK)], idx_d)
        pltpu.async_copy(xs_hbm.at[idx_s], rows_v, sem).wait()
        pltpu.sync_copy(rows_v, acc.at[idx_d], add=True)
        return carry
    lax.fori_loop(0, _CPW, eloop, 0)
    plsc.subcore_barrier()

    def wloop(i, carry):
        r0 = s * _RPS + i * 64
        pltpu.sync_copy(acc.at[pl.ds(r0, 64)], out_hbm.at[c, pl.ds(r0, 64)])
        return carry
    lax.fori_loop(0, _RPS // 64, wloop, 0)


_sc_scatter = pl.kernel(
    _sc_scatter_body,
    out_type=jax.ShapeDtypeStruct((_NCORE, _NPAD, _H), jnp.float32),
    mesh=plsc.VectorSubcoreMesh(core_axis_name="c", subcore_axis_name="s"),
    scratch_types=[
        pltpu.VMEM((_CHUNK,), jnp.int32),
        pltpu.VMEM((_CHUNK,), jnp.int32),
        pltpu.VMEM((_CHUNK, _H), jnp.float32),
        pltpu.VMEM((16, _H), jnp.float32),
        pltpu.VMEM_SHARED((_NPAD, _H), jnp.float32),
        pltpu.SemaphoreType.DMA,
    ],
)


def _tc_pre_body(x_ref, degp_ref, w_ref, xs_ref, dinv_ref):
    deg = degp_ref[0, : _N, 0] + degp_ref[1, : _N, 0] + 1.0
    dinv = lax.rsqrt(deg).reshape(_N, 1)
    dinv_ref[...] = dinv
    xs_ref[...] = jnp.dot(x_ref[...], w_ref[...],
                          preferred_element_type=jnp.float32) * dinv


_tc_pre = pl.pallas_call(
    _tc_pre_body,
    out_shape=(jax.ShapeDtypeStruct((_N, _H), jnp.float32),
               jax.ShapeDtypeStruct((_N, 1), jnp.float32)),
)


def _gcn_post(aggp_ref, xs_ref, dinv_ref, b_ref, g_ref, be_ref):
    agg = aggp_ref[0, : _N, :] + aggp_ref[1, : _N, :] + xs_ref[...]
    pre = agg * dinv_ref[...] + b_ref[...]
    mu = jnp.mean(pre, axis=0, keepdims=True)
    xc = pre - mu
    var = jnp.mean(xc * xc, axis=0, keepdims=True)
    return jnp.maximum(xc * lax.rsqrt(var + _EPS) * g_ref[...] + be_ref[...],
                       0.0)


def _tc_mid_body(aggp_ref, xs_ref, dinv_ref, b_ref, g_ref, be_ref, w_ref,
                 out_ref):
    h = _gcn_post(aggp_ref, xs_ref, dinv_ref, b_ref, g_ref, be_ref)
    out_ref[...] = jnp.dot(h, w_ref[...],
                           preferred_element_type=jnp.float32) * dinv_ref[...]


_tc_mid = pl.pallas_call(
    _tc_mid_body,
    out_shape=jax.ShapeDtypeStruct((_N, _H), jnp.float32),
)


def _tc_post_body(aggp_ref, xs_ref, dinv_ref, b_ref, g_ref, be_ref,
                  batch_ref, wc_ref, bc_ref, out_ref):
    h = _gcn_post(aggp_ref, xs_ref, dinv_ref, b_ref, g_ref, be_ref)
    gids = lax.broadcasted_iota(jnp.int32, (_G, _N), 0)
    onehot = (gids == batch_ref[...]).astype(jnp.float32)
    sums = jnp.dot(onehot, h, preferred_element_type=jnp.float32)
    counts = jnp.sum(onehot, axis=1, keepdims=True)
    pooled = sums / jnp.maximum(counts, 1.0)
    out_ref[...] = jnp.dot(pooled, wc_ref[...],
                           preferred_element_type=jnp.float32) + bc_ref[...]


_tc_post = pl.pallas_call(
    _tc_post_body,
    out_shape=jax.ShapeDtypeStruct((_G, _C), jnp.float32),
)


def kernel(node_features, edge_index, batch, W0, b0, g0, be0, W1, b1, g1,
           be1, W2, b2, g2, be2, Wc, bc):
    src = edge_index[0]
    dst = edge_index[1]
    npad = _EPAD - _E
    src_p = jnp.concatenate([src, jnp.zeros((npad,), jnp.int32)])
    dst_p = jnp.concatenate([dst, jnp.full((npad,), _N, jnp.int32)])

    degp = _sc_degree(dst_p)
    xs0, dinv = _tc_pre(node_features, degp, W0)

    r1 = lambda v: v.reshape(1, -1)
    agg0 = _sc_scatter(xs0, src_p, dst_p)
    xs1 = _tc_mid(agg0, xs0, dinv, r1(b0), r1(g0), r1(be0), W1)
    agg1 = _sc_scatter(xs1, src_p, dst_p)
    xs2 = _tc_mid(agg1, xs1, dinv, r1(b1), r1(g1), r1(be1), W2)
    agg2 = _sc_scatter(xs2, src_p, dst_p)
    return _tc_post(agg2, xs2, dinv, r1(b2), r1(g2), r1(be2),
                    batch.reshape(1, _N), Wc, r1(bc))


# speedup vs baseline: 1.0002x; 1.0002x over previous
import jax
import jax.numpy as jnp
from jax import lax
from jax.experimental import pallas as pl
from jax.experimental.pallas import tpu as pltpu
from jax.experimental.pallas import tpu_sc as plsc

_N = 10000
_E = 320000
_D = 128
_H = 128
_C = 3
_G = 64
_EPS = 1e-5

_NSUB = 16
_NCORE = 2
_NW = _NSUB * _NCORE
_CHUNK = 128
_CPW = 80
_EPAD = _NW * _CPW * _CHUNK
_NPAD = 10240
_RPS = _NPAD // _NSUB


def _sc_degree_body(dst_hbm, out_hbm, idx_d, ones_v, zbuf, acc, sem):
    c = lax.axis_index("c")
    s = lax.axis_index("s")
    wid = c * _NSUB + s
    one16 = jnp.ones((16,), jnp.float32)
    zero16 = jnp.zeros((16,), jnp.float32)
    for r in range(16):
        zbuf[r] = zero16
    for r in range(_CHUNK):
        ones_v[r] = one16

    def zloop(i, carry):
        pltpu.sync_copy(zbuf, acc.at[pl.ds(s * _RPS + i * 16, 16)])
        return carry
    lax.fori_loop(0, _RPS // 16, zloop, 0)
    plsc.subcore_barrier()

    def eloop(i, carry):
        base = (wid * _CPW + i) * _CHUNK
        pltpu.sync_copy(dst_hbm.at[pl.ds(base, _CHUNK)], idx_d)
        pltpu.sync_copy(ones_v, acc.at[idx_d], add=True)
        return carry
    lax.fori_loop(0, _CPW, eloop, 0)
    plsc.subcore_barrier()

    pltpu.sync_copy(acc.at[pl.ds(s * _RPS, _RPS)],
                    out_hbm.at[c, pl.ds(s * _RPS, _RPS)])


_sc_degree = pl.kernel(
    _sc_degree_body,
    out_type=jax.ShapeDtypeStruct((_NCORE, _NPAD, 16), jnp.float32),
    mesh=plsc.VectorSubcoreMesh(core_axis_name="c", subcore_axis_name="s"),
    scratch_types=[
        pltpu.VMEM((_CHUNK,), jnp.int32),
        pltpu.VMEM((_CHUNK, 16), jnp.float32),
        pltpu.VMEM((16, 16), jnp.float32),
        pltpu.VMEM_SHARED((_NPAD, 16), jnp.float32),
        pltpu.SemaphoreType.DMA,
    ],
)


def _sc_scatter_body(xs_hbm, src_hbm, dst_hbm, out_hbm,
                     idx_s, idx_d, rows_v, zbuf, acc, sem):
    c = lax.axis_index("c")
    s = lax.axis_index("s")
    wid = c * _NSUB + s
    zero16 = jnp.zeros((16,), jnp.float32)
    for r in range(16):
        for j in range(_H // 16):
            zbuf[r, pl.ds(j * 16, 16)] = zero16

    def zloop(i, carry):
        pltpu.sync_copy(zbuf, acc.at[pl.ds(s * _RPS + i * 16, 16)])
        return carry
    lax.fori_loop(0, _RPS // 16, zloop, 0)
    plsc.subcore_barrier()

    def eloop(i, carry):
        base = (wid * _CPW + i) * _CHUNK
        pltpu.sync_copy(src_hbm.at[pl.ds(base, _CHUNK)], idx_s)
        pltpu.sync_copy(dst_hbm.at[pl.ds(base, _CHUNK)], idx_d)
        pltpu.async_copy(xs_hbm.at[idx_s], rows_v, sem).wait()
        pltpu.sync_copy(rows_v, acc.at[idx_d], add=True)
        return carry
    lax.fori_loop(0, _CPW, eloop, 0)
    plsc.subcore_barrier()

    def wloop(i, carry):
        r0 = s * _RPS + i * 64
        pltpu.sync_copy(acc.at[pl.ds(r0, 64)], out_hbm.at[c, pl.ds(r0, 64)])
        return carry
    lax.fori_loop(0, _RPS // 64, wloop, 0)


_sc_scatter = pl.kernel(
    _sc_scatter_body,
    out_type=jax.ShapeDtypeStruct((_NCORE, _NPAD, _H), jnp.float32),
    mesh=plsc.VectorSubcoreMesh(core_axis_name="c", subcore_axis_name="s"),
    scratch_types=[
        pltpu.VMEM((_CHUNK,), jnp.int32),
        pltpu.VMEM((_CHUNK,), jnp.int32),
        pltpu.VMEM((_CHUNK, _H), jnp.float32),
        pltpu.VMEM((16, _H), jnp.float32),
        pltpu.VMEM_SHARED((_NPAD, _H), jnp.float32),
        pltpu.SemaphoreType.DMA,
    ],
)


def _tc_pre_body(x_ref, degp_ref, w_ref, xs_ref, dinv_ref):
    deg = degp_ref[0, : _N, 0] + degp_ref[1, : _N, 0] + 1.0
    dinv = lax.rsqrt(deg).reshape(_N, 1)
    dinv_ref[...] = dinv
    xs_ref[...] = jnp.dot(x_ref[...], w_ref[...],
                          preferred_element_type=jnp.float32) * dinv


_tc_pre = pl.pallas_call(
    _tc_pre_body,
    out_shape=(jax.ShapeDtypeStruct((_N, _H), jnp.float32),
               jax.ShapeDtypeStruct((_N, 1), jnp.float32)),
)


def _gcn_post(aggp_ref, xs_ref, dinv_ref, b_ref, g_ref, be_ref):
    agg = aggp_ref[0, : _N, :] + aggp_ref[1, : _N, :] + xs_ref[...]
    pre = agg * dinv_ref[...] + b_ref[...]
    mu = jnp.mean(pre, axis=0, keepdims=True)
    xc = pre - mu
    var = jnp.mean(xc * xc, axis=0, keepdims=True)
    return jnp.maximum(xc * lax.rsqrt(var + _EPS) * g_ref[...] + be_ref[...],
                       0.0)


def _tc_mid_body(aggp_ref, xs_ref, dinv_ref, b_ref, g_ref, be_ref, w_ref,
                 out_ref):
    h = _gcn_post(aggp_ref, xs_ref, dinv_ref, b_ref, g_ref, be_ref)
    out_ref[...] = jnp.dot(h, w_ref[...],
                           preferred_element_type=jnp.float32) * dinv_ref[...]


_tc_mid = pl.pallas_call(
    _tc_mid_body,
    out_shape=jax.ShapeDtypeStruct((_N, _H), jnp.float32),
)


def _tc_post_body(aggp_ref, xs_ref, dinv_ref, b_ref, g_ref, be_ref,
                  batch_ref, wc_ref, bc_ref, out_ref):
    h = _gcn_post(aggp_ref, xs_ref, dinv_ref, b_ref, g_ref, be_ref)
    gids = lax.broadcasted_iota(jnp.int32, (_G, _N), 0)
    onehot = (gids == batch_ref[...]).astype(jnp.float32)
    sums = jnp.dot(onehot, h, preferred_element_type=jnp.float32)
    counts = jnp.sum(onehot, axis=1, keepdims=True)
    pooled = sums / jnp.maximum(counts, 1.0)
    out_ref[...] = jnp.dot(pooled, wc_ref[...],
                           preferred_element_type=jnp.float32) + bc_ref[...]


_tc_post = pl.pallas_call(
    _tc_post_body,
    out_shape=jax.ShapeDtypeStruct((_G, _C), jnp.float32),
)


def kernel(node_features, edge_index, batch, W0, b0, g0, be0, W1, b1, g1,
           be1, W2, b2, g2, be2, Wc, bc):
    src = edge_index[0]
    dst = edge_index[1]
    npad = _EPAD - _E
    src_p = jnp.concatenate([src, jnp.zeros((npad,), jnp.int32)])
    dummy = _N + jnp.arange(npad, dtype=jnp.int32) % (_NPAD - _N)
    dst_p = jnp.concatenate([dst, dummy])

    degp = _sc_degree(dst_p)
    xs0, dinv = _tc_pre(node_features, degp, W0)

    r1 = lambda v: v.reshape(1, -1)
    agg0 = _sc_scatter(xs0, src_p, dst_p)
    xs1 = _tc_mid(agg0, xs0, dinv, r1(b0), r1(g0), r1(be0), W1)
    agg1 = _sc_scatter(xs1, src_p, dst_p)
    xs2 = _tc_mid(agg1, xs1, dinv, r1(b1), r1(g1), r1(be1), W2)
    agg2 = _sc_scatter(xs2, src_p, dst_p)
    return _tc_post(agg2, xs2, dinv, r1(b2), r1(g2), r1(be2),
                    batch.reshape(1, _N), Wc, r1(bc))


# speedup vs baseline: 2.2513x; 2.2507x over previous
import jax
import jax.numpy as jnp
from jax import lax
from jax.experimental import pallas as pl
from jax.experimental.pallas import tpu as pltpu
from jax.experimental.pallas import tpu_sc as plsc

_N = 10000
_E = 320000
_D = 128
_H = 128
_C = 3
_G = 64
_EPS = 1e-5

_NSUB = 16
_NCORE = 2
_NW = _NSUB * _NCORE
_CHUNK = 128
_CPW = 80
_EPAD = _NW * _CPW * _CHUNK
_NPAD = 10240
_RPS = _NPAD // _NSUB


def _sc_degree_body(dst_hbm, out_hbm, idx_d, ones_v, zbuf, acc, sem):
    c = lax.axis_index("c")
    s = lax.axis_index("s")
    wid = c * _NSUB + s
    one16 = jnp.ones((16,), jnp.float32)
    zero16 = jnp.zeros((16,), jnp.float32)
    for r in range(16):
        zbuf[r] = zero16
    for r in range(_CHUNK):
        ones_v[r] = one16

    def zloop(i, carry):
        pltpu.sync_copy(zbuf, acc.at[pl.ds(s * _RPS + i * 16, 16)])
        return carry
    lax.fori_loop(0, _RPS // 16, zloop, 0)
    plsc.subcore_barrier()

    def eloop(i, carry):
        base = (wid * _CPW + i) * _CHUNK
        pltpu.sync_copy(dst_hbm.at[pl.ds(base, _CHUNK)], idx_d)
        pltpu.sync_copy(ones_v, acc.at[idx_d], add=True)
        return carry
    lax.fori_loop(0, _CPW, eloop, 0)
    plsc.subcore_barrier()

    pltpu.sync_copy(acc.at[pl.ds(s * _RPS, _RPS)],
                    out_hbm.at[c, pl.ds(s * _RPS, _RPS)])


_sc_degree = pl.kernel(
    _sc_degree_body,
    out_type=jax.ShapeDtypeStruct((_NCORE, _NPAD, 16), jnp.float32),
    mesh=plsc.VectorSubcoreMesh(core_axis_name="c", subcore_axis_name="s"),
    scratch_types=[
        pltpu.VMEM((_CHUNK,), jnp.int32),
        pltpu.VMEM((_CHUNK, 16), jnp.float32),
        pltpu.VMEM((16, 16), jnp.float32),
        pltpu.VMEM_SHARED((_NPAD, 16), jnp.float32),
        pltpu.SemaphoreType.DMA,
    ],
)


def _sc_scatter_body(xs_hbm, src_hbm, dst_hbm, out_hbm,
                     idx_s, idx_d, rows_v, zbuf, acc, sem):
    c = lax.axis_index("c")
    s = lax.axis_index("s")
    wid = c * _NSUB + s
    zero16 = jnp.zeros((16,), jnp.float32)
    for r in range(16):
        for j in range(_H // 16):
            zbuf[r, pl.ds(j * 16, 16)] = zero16

    def zloop(i, carry):
        pltpu.sync_copy(zbuf, acc.at[pl.ds(s * _RPS + i * 16, 16)])
        return carry
    lax.fori_loop(0, _RPS // 16, zloop, 0)
    plsc.subcore_barrier()

    def eloop(i, carry):
        base = (wid * _CPW + i) * _CHUNK
        pltpu.sync_copy(src_hbm.at[pl.ds(base, _CHUNK)], idx_s)
        pltpu.sync_copy(dst_hbm.at[pl.ds(base, _CHUNK)], idx_d)
        pltpu.async_copy(xs_hbm.at[idx_s], rows_v, sem).wait()
        pltpu.sync_copy(rows_v, acc.at[idx_d], add=True)
        return carry
    lax.fori_loop(0, _CPW, eloop, 0)
    plsc.subcore_barrier()

    def wloop(i, carry):
        r0 = s * _RPS + i * 64
        pltpu.sync_copy(acc.at[pl.ds(r0, 64)], out_hbm.at[c, pl.ds(r0, 64)])
        return carry
    lax.fori_loop(0, _RPS // 64, wloop, 0)


_sc_scatter = pl.kernel(
    _sc_scatter_body,
    out_type=jax.ShapeDtypeStruct((_NCORE, _NPAD, _H), jnp.float32),
    mesh=plsc.VectorSubcoreMesh(core_axis_name="c", subcore_axis_name="s"),
    scratch_types=[
        pltpu.VMEM((_CHUNK,), jnp.int32),
        pltpu.VMEM((_CHUNK,), jnp.int32),
        pltpu.VMEM((_CHUNK, _H), jnp.float32),
        pltpu.VMEM((16, _H), jnp.float32),
        pltpu.VMEM_SHARED((_NPAD, _H), jnp.float32),
        pltpu.SemaphoreType.DMA,
    ],
)


def _tc_pre_body(x_ref, degp_ref, w_ref, xs_ref, dinv_ref):
    deg = degp_ref[0, : _N, 0] + degp_ref[1, : _N, 0] + 1.0
    dinv = lax.rsqrt(deg).reshape(_N, 1)
    dinv_ref[...] = dinv
    xs_ref[...] = jnp.dot(x_ref[...], w_ref[...],
                          preferred_element_type=jnp.float32) * dinv


_tc_pre = pl.pallas_call(
    _tc_pre_body,
    out_shape=(jax.ShapeDtypeStruct((_N, _H), jnp.float32),
               jax.ShapeDtypeStruct((_N, 1), jnp.float32)),
)


def _gcn_post(aggp_ref, xs_ref, dinv_ref, b_ref, g_ref, be_ref):
    agg = aggp_ref[0, : _N, :] + aggp_ref[1, : _N, :] + xs_ref[...]
    pre = agg * dinv_ref[...] + b_ref[...]
    mu = jnp.mean(pre, axis=0, keepdims=True)
    xc = pre - mu
    var = jnp.mean(xc * xc, axis=0, keepdims=True)
    return jnp.maximum(xc * lax.rsqrt(var + _EPS) * g_ref[...] + be_ref[...],
                       0.0)


def _tc_mid_body(aggp_ref, xs_ref, dinv_ref, b_ref, g_ref, be_ref, w_ref,
                 out_ref):
    h = _gcn_post(aggp_ref, xs_ref, dinv_ref, b_ref, g_ref, be_ref)
    out_ref[...] = jnp.dot(h, w_ref[...],
                           preferred_element_type=jnp.float32) * dinv_ref[...]


_tc_mid = pl.pallas_call(
    _tc_mid_body,
    out_shape=jax.ShapeDtypeStruct((_N, _H), jnp.float32),
)


def _tc_post_body(aggp_ref, xs_ref, dinv_ref, b_ref, g_ref, be_ref,
                  batch_ref, wc_ref, bc_ref, out_ref):
    h = _gcn_post(aggp_ref, xs_ref, dinv_ref, b_ref, g_ref, be_ref)
    gids = lax.broadcasted_iota(jnp.int32, (_G, _N), 0)
    onehot = (gids == batch_ref[...]).astype(jnp.float32)
    sums = jnp.dot(onehot, h, preferred_element_type=jnp.float32)
    counts = jnp.sum(onehot, axis=1, keepdims=True)
    pooled = sums / jnp.maximum(counts, 1.0)
    out_ref[...] = jnp.dot(pooled, wc_ref[...],
                           preferred_element_type=jnp.float32) + bc_ref[...]


_tc_post = pl.pallas_call(
    _tc_post_body,
    out_shape=jax.ShapeDtypeStruct((_G, _C), jnp.float32),
)


def kernel(node_features, edge_index, batch, W0, b0, g0, be0, W1, b1, g1,
           be1, W2, b2, g2, be2, Wc, bc):
    src = edge_index[0]
    dst = edge_index[1]
    npad = _EPAD - _E
    pad_iota = jnp.arange(npad, dtype=jnp.int32)
    src_p = jnp.concatenate([src, pad_iota % _N])
    dst_p = jnp.concatenate([dst, _N + pad_iota % (_NPAD - _N)])

    degp = _sc_degree(dst_p)
    xs0, dinv = _tc_pre(node_features, degp, W0)

    r1 = lambda v: v.reshape(1, -1)
    agg0 = _sc_scatter(xs0, src_p, dst_p)
    xs1 = _tc_mid(agg0, xs0, dinv, r1(b0), r1(g0), r1(be0), W1)
    agg1 = _sc_scatter(xs1, src_p, dst_p)
    xs2 = _tc_mid(agg1, xs1, dinv, r1(b1), r1(g1), r1(be1), W2)
    agg2 = _sc_scatter(xs2, src_p, dst_p)
    return _tc_post(agg2, xs2, dinv, r1(b2), r1(g2), r1(be2),
                    batch.reshape(1, _N), Wc, r1(bc))


# speedup vs baseline: 2.9633x; 1.3163x over previous
import jax
import jax.numpy as jnp
from jax import lax
from jax.experimental import pallas as pl
from jax.experimental.pallas import tpu as pltpu
from jax.experimental.pallas import tpu_sc as plsc

_N = 10000
_E = 320000
_D = 128
_H = 128
_C = 3
_G = 64
_EPS = 1e-5

_NSUB = 16
_NCORE = 2
_NW = _NSUB * _NCORE
_CHUNK = 128
_CPW = 80
_EPAD = _NW * _CPW * _CHUNK
_EXTRA = 2 * _CHUNK
_NPAD = 10240
_RPS = _NPAD // _NSUB


def _sc_degree_body(dst_hbm, out_hbm, idx_d, ones_v, zbuf, acc, sem):
    c = lax.axis_index("c")
    s = lax.axis_index("s")
    wid = c * _NSUB + s
    one16 = jnp.ones((16,), jnp.float32)
    zero16 = jnp.zeros((16,), jnp.float32)
    for r in range(16):
        zbuf[r] = zero16
    for r in range(_CHUNK):
        ones_v[r] = one16

    def zloop(i, carry):
        pltpu.sync_copy(zbuf, acc.at[pl.ds(s * _RPS + i * 16, 16)])
        return carry
    lax.fori_loop(0, _RPS // 16, zloop, 0)
    plsc.subcore_barrier()

    def eloop(i, carry):
        base = (wid * _CPW + i) * _CHUNK
        pltpu.sync_copy(dst_hbm.at[pl.ds(base, _CHUNK)], idx_d)
        pltpu.sync_copy(ones_v, acc.at[idx_d], add=True)
        return carry
    lax.fori_loop(0, _CPW, eloop, 0)
    plsc.subcore_barrier()

    pltpu.sync_copy(acc.at[pl.ds(s * _RPS, _RPS)],
                    out_hbm.at[c, pl.ds(s * _RPS, _RPS)])


_sc_degree = pl.kernel(
    _sc_degree_body,
    out_type=jax.ShapeDtypeStruct((_NCORE, _NPAD, 16), jnp.float32),
    mesh=plsc.VectorSubcoreMesh(core_axis_name="c", subcore_axis_name="s"),
    scratch_types=[
        pltpu.VMEM((_CHUNK,), jnp.int32),
        pltpu.VMEM((_CHUNK, 16), jnp.float32),
        pltpu.VMEM((16, 16), jnp.float32),
        pltpu.VMEM_SHARED((_NPAD, 16), jnp.float32),
        pltpu.SemaphoreType.DMA,
    ],
)


def _sc_scatter_body(xs_hbm, src_hbm, dst_hbm, out_hbm,
                     idxs0, idxs1, idxs2, idxs3, idxd0, idxd1, idxd2, idxd3,
                     rows0, rows1, zbuf, acc,
                     semi0, semi1, semi2, semi3, semg0, semg1):
    c = lax.axis_index("c")
    s = lax.axis_index("s")
    wid = c * _NSUB + s
    cbase = wid * _CPW
    idxs = (idxs0, idxs1, idxs2, idxs3)
    idxd = (idxd0, idxd1, idxd2, idxd3)
    rows = (rows0, rows1)
    semi = (semi0, semi1, semi2, semi3)
    semg = (semg0, semg1)

    zero16 = jnp.zeros((16,), jnp.float32)
    for r in range(16):
        for j in range(_H // 16):
            zbuf[r, pl.ds(j * 16, 16)] = zero16

    def start_idx(q, ci):
        pltpu.async_copy(src_hbm.at[pl.ds(ci * _CHUNK, _CHUNK)], idxs[q],
                         semi[q])
        pltpu.async_copy(dst_hbm.at[pl.ds(ci * _CHUNK, _CHUNK)], idxd[q],
                         semi[q])

    def wait_idx(q):
        pltpu.make_async_copy(src_hbm.at[pl.ds(0, _CHUNK)], idxs[q],
                              semi[q]).wait()
        pltpu.make_async_copy(src_hbm.at[pl.ds(0, _CHUNK)], idxd[q],
                              semi[q]).wait()

    def start_gather(b, q):
        pltpu.async_copy(xs_hbm.at[idxs[q]], rows[b], semg[b])

    def wait_gather(b, q):
        pltpu.make_async_copy(xs_hbm.at[idxs[q]], rows[b], semg[b]).wait()

    pltpu.sync_copy(src_hbm.at[pl.ds(cbase * _CHUNK, _CHUNK)], idxs0)
    pltpu.sync_copy(dst_hbm.at[pl.ds(cbase * _CHUNK, _CHUNK)], idxd0)
    start_gather(0, 0)
    start_idx(1, cbase + 1)

    def zloop(i, carry):
        pltpu.sync_copy(zbuf, acc.at[pl.ds(s * _RPS + i * 16, 16)])
        return carry
    lax.fori_loop(0, _RPS // 16, zloop, 0)
    plsc.subcore_barrier()

    def step(i, b, q):
        ob = 1 - b
        wait_gather(b, q)
        pltpu.sync_copy(rows[b], acc.at[idxd[q]], add=True)
        start_idx((q + 2) % 4, i + 2)
        wait_idx((q + 1) % 4)
        start_gather(ob, (q + 1) % 4)

    step(cbase + 0, 0, 0)
    step(cbase + 1, 1, 1)
    step(cbase + 2, 0, 2)
    step(cbase + 3, 1, 3)

    def quad(j, carry):
        i0 = cbase + 4 * j
        step(i0 + 0, 0, 0)
        step(i0 + 1, 1, 1)
        step(i0 + 2, 0, 2)
        step(i0 + 3, 1, 3)
        return carry
    lax.fori_loop(1, _CPW // 4, quad, 0)

    wait_gather(0, 0)
    wait_idx(1)

    plsc.subcore_barrier()
    pltpu.sync_copy(acc.at[pl.ds(s * _RPS, _RPS)],
                    out_hbm.at[c, pl.ds(s * _RPS, _RPS)])


_sc_scatter = pl.kernel(
    _sc_scatter_body,
    out_type=jax.ShapeDtypeStruct((_NCORE, _NPAD, _H), jnp.float32),
    mesh=plsc.VectorSubcoreMesh(core_axis_name="c", subcore_axis_name="s"),
    scratch_types=(
        [pltpu.VMEM((_CHUNK,), jnp.int32)] * 8
        + [pltpu.VMEM((_CHUNK, _H), jnp.float32)] * 2
        + [pltpu.VMEM((16, _H), jnp.float32),
           pltpu.VMEM_SHARED((_NPAD, _H), jnp.float32)]
        + [pltpu.SemaphoreType.DMA] * 6
    ),
)


def _tc_pre_body(x_ref, degp_ref, w_ref, xs_ref, dinv_ref):
    deg = degp_ref[0, : _N, 0] + degp_ref[1, : _N, 0] + 1.0
    dinv = lax.rsqrt(deg).reshape(_N, 1)
    dinv_ref[...] = dinv
    xs_ref[...] = jnp.dot(x_ref[...], w_ref[...],
                          preferred_element_type=jnp.float32) * dinv


_tc_pre = pl.pallas_call(
    _tc_pre_body,
    out_shape=(jax.ShapeDtypeStruct((_N, _H), jnp.float32),
               jax.ShapeDtypeStruct((_N, 1), jnp.float32)),
)


def _gcn_post(aggp_ref, xs_ref, dinv_ref, b_ref, g_ref, be_ref):
    agg = aggp_ref[0, : _N, :] + aggp_ref[1, : _N, :] + xs_ref[...]
    pre = agg * dinv_ref[...] + b_ref[...]
    mu = jnp.mean(pre, axis=0, keepdims=True)
    xc = pre - mu
    var = jnp.mean(xc * xc, axis=0, keepdims=True)
    return jnp.maximum(xc * lax.rsqrt(var + _EPS) * g_ref[...] + be_ref[...],
                       0.0)


def _tc_mid_body(aggp_ref, xs_ref, dinv_ref, b_ref, g_ref, be_ref, w_ref,
                 out_ref):
    h = _gcn_post(aggp_ref, xs_ref, dinv_ref, b_ref, g_ref, be_ref)
    out_ref[...] = jnp.dot(h, w_ref[...],
                           preferred_element_type=jnp.float32) * dinv_ref[...]


_tc_mid = pl.pallas_call(
    _tc_mid_body,
    out_shape=jax.ShapeDtypeStruct((_N, _H), jnp.float32),
)


def _tc_post_body(aggp_ref, xs_ref, dinv_ref, b_ref, g_ref, be_ref,
                  batch_ref, wc_ref, bc_ref, out_ref):
    h = _gcn_post(aggp_ref, xs_ref, dinv_ref, b_ref, g_ref, be_ref)
    gids = lax.broadcasted_iota(jnp.int32, (_G, _N), 0)
    onehot = (gids == batch_ref[...]).astype(jnp.float32)
    sums = jnp.dot(onehot, h, preferred_element_type=jnp.float32)
    counts = jnp.sum(onehot, axis=1, keepdims=True)
    pooled = sums / jnp.maximum(counts, 1.0)
    out_ref[...] = jnp.dot(pooled, wc_ref[...],
                           preferred_element_type=jnp.float32) + bc_ref[...]


_tc_post = pl.pallas_call(
    _tc_post_body,
    out_shape=jax.ShapeDtypeStruct((_G, _C), jnp.float32),
)


def kernel(node_features, edge_index, batch, W0, b0, g0, be0, W1, b1, g1,
           be1, W2, b2, g2, be2, Wc, bc):
    src = edge_index[0]
    dst = edge_index[1]
    npad = _EPAD + _EXTRA - _E
    pad_iota = jnp.arange(npad, dtype=jnp.int32)
    src_p = jnp.concatenate([src, pad_iota % _N])
    dst_p = jnp.concatenate([dst, _N + pad_iota % (_NPAD - _N)])

    degp = _sc_degree(dst_p)
    xs0, dinv = _tc_pre(node_features, degp, W0)

    r1 = lambda v: v.reshape(1, -1)
    agg0 = _sc_scatter(xs0, src_p, dst_p)
    xs1 = _tc_mid(agg0, xs0, dinv, r1(b0), r1(g0), r1(be0), W1)
    agg1 = _sc_scatter(xs1, src_p, dst_p)
    xs2 = _tc_mid(agg1, xs1, dinv, r1(b1), r1(g1), r1(be1), W2)
    agg2 = _sc_scatter(xs2, src_p, dst_p)
    return _tc_post(agg2, xs2, dinv, r1(b2), r1(g2), r1(be2),
                    batch.reshape(1, _N), Wc, r1(bc))
